# baseline (speedup 1.0000x reference)
import jax
import jax.numpy as jnp
from jax import lax
from jax.experimental import pallas as pl

_B, _H, _W, _C = 128, 32, 32, 96
_N = _H * _W
_KEEP = _N // 2
_RECIP96 = 0.010416666977107525

_ERF_P = (7.85386146e-05, -0.000801019371, 0.00518832775, -0.0268538129,
          0.112835854, -0.37612626, 1.12837911)
_ERFC_P = (0.0232682, -0.138703942, 0.368742466, -0.582473278, 0.621000469,
           -0.494451523, 0.340488, -0.274112701, 0.563825965)
_ERFC_R = (-10.477664, 12.9772, -7.49551868, 2.92101908, -1.01526523,
           0.42184633, -0.282076746, 0.564189494)


def _horner(r, coeffs):
    p = r * coeffs[0] + coeffs[1]
    for c in coeffs[2:]:
        p = p * r + c
    return p


def _erfc_xla(x):
    ax = jnp.abs(x)
    x2 = x * x
    small = 1.0 - x * _horner(x2, _ERF_P)
    nx2 = -x2
    z = jnp.exp(nx2)
    zq = z * (1.0 / ax)
    r = 1.0 / x2
    p = jnp.where(ax < 2.0, _horner(r, _ERFC_P), _horner(r, _ERFC_R))
    y = zq * p
    y = jnp.where(nx2 < -88.7228394, 0.0, y)
    y = jnp.where(x < 0.0, 2.0 - y, y)
    return jnp.where(ax < 1.0, small, y)


def _gelu_exact(x):
    return (0.5 * x) * _erfc_xla(-x * 0.7071067811865476)


def _tree96(v):
    gs = []
    for g in range(3):
        a = v[32 * g: 32 * g + 8]
        for r in range(1, 4):
            a = a + v[32 * g + 8 * r: 32 * g + 8 * r + 8]
        u = a[0:4] + a[4:8]
        w = u[0:2] + u[2:4]
        gs.append(w[0:1] + w[1:2])
    return (gs[0] + gs[1]) + gs[2]


def _make_mlp_body():
    def body(x_ref, m_ref, g_ref, lng_ref, lnb_ref, w1_ref, b1_ref, w2_ref,
             b2_ref, w3_ref, b3_ref, w4_ref, b4_ref, score_ref, nm_ref):
        x = x_ref[0] * m_ref[0]
        xt = jnp.transpose(x)
        mu = _tree96(xt) * _RECIP96
        cent = xt - mu
        var = _tree96(cent * cent) * _RECIP96
        y = cent / jnp.sqrt(var + 1e-5) * lng_ref[...] + lnb_ref[...]
        h1 = _gelu_exact(jnp.dot(w1_ref[...], y) + b1_ref[...])
        gin = h1[_C // 2:, :]
        acc = gin[:, 0:128]
        for ch in range(1, 8):
            acc = acc + gin[:, 128 * ch: 128 * (ch + 1)]
        gm = jnp.sum(acc, axis=1, keepdims=True) * (1.0 / 1024.0)
        xcat = jnp.concatenate(
            [h1[: _C // 2, :], jnp.broadcast_to(gm, (_C // 2, _N))], axis=0)
        h2 = _gelu_exact(jnp.dot(w2_ref[...], xcat) + b2_ref[...])
        h3 = _gelu_exact(jnp.dot(w3_ref[...], h2) + b3_ref[...])
        logits = jnp.dot(w4_ref[...], h3) + b4_ref[...]
        mx = jnp.maximum(logits[0:1], logits[1:2])
        sh = logits - mx
        e = jnp.exp(sh)
        lse = jnp.log(e[0:1] + e[1:2])
        pred = sh - lse
        score_ref[0] = pred[0:1]
        vv = pred + g_ref[0]
        mv = jnp.maximum(vv[0:1], vv[1:2])
        e2 = jnp.exp(vv - mv)
        nm_ref[0] = e2[0:1] / (e2[0:1] + e2[1:2])
    return body


def _run_mlp(xr, mr, gnoise_t, ln_g, ln_b, W1, b1, W2, b2, W3, b3, W4, b4):
    row3 = lambda b: (b, 0, 0)
    full = lambda b: (0, 0)
    return pl.pallas_call(
        _make_mlp_body(),
        grid=(_B,),
        in_specs=[
            pl.BlockSpec((1, _N, _C), row3),
            pl.BlockSpec((1, _N, 1), row3),
            pl.BlockSpec((1, 2, _N), row3),
            pl.BlockSpec((_C, 1), full),
            pl.BlockSpec((_C, 1), full),
            pl.BlockSpec((_C, _C), full),
            pl.BlockSpec((_C, 1), full),
            pl.BlockSpec((_C // 2, _C), full),
            pl.BlockSpec((_C // 2, 1), full),
            pl.BlockSpec((_C // 4, _C // 2), full),
            pl.BlockSpec((_C // 4, 1), full),
            pl.BlockSpec((2, _C // 4), full),
            pl.BlockSpec((2, 1), full),
        ],
        out_specs=[
            pl.BlockSpec((1, 1, _N), row3),
            pl.BlockSpec((1, 1, _N), row3),
        ],
        out_shape=[
            jax.ShapeDtypeStruct((_B, 1, _N), jnp.float32),
            jax.ShapeDtypeStruct((_B, 1, _N), jnp.float32),
        ],
    )(xr, mr, gnoise_t,
      ln_g.reshape(_C, 1), ln_b.reshape(_C, 1),
      W1, b1.reshape(_C, 1),
      W2, b2.reshape(_C // 2, 1),
      W3, b3.reshape(_C // 4, 1),
      W4, b4.reshape(2, 1))


def _sort_body(s_ref, ks_ref, is_ref):
    s = s_ref[...]
    u = lax.bitcast_convert_type(s, jnp.uint32)
    kmono = jnp.where(u >= jnp.uint32(0x80000000), ~u,
                      u | jnp.uint32(0x80000000))
    key = ~kmono
    idx = lax.broadcasted_iota(jnp.int32, (_N, _B), 0)
    row = lax.broadcasted_iota(jnp.int32, (_N, 1), 0)

    def roll0(x, sh):
        sh = sh % _N
        return jnp.concatenate([x[sh:], x[:sh]], axis=0)

    for stage in range(10):
        for sub in range(stage, -1, -1):
            d = 1 << sub
            self_lo = (row & d) == 0
            dir_up = (row & (2 << stage)) == 0
            kp = jnp.where(self_lo, roll0(key, d), roll0(key, -d))
            ip = jnp.where(self_lo, roll0(idx, d), roll0(idx, -d))
            cmp = (key > kp) | ((key == kp) & (idx > ip))
            g_pair = cmp ^ (~self_lo)
            swap = jnp.logical_not(g_pair ^ dir_up)
            key = jnp.where(swap, kp, key)
            idx = jnp.where(swap, ip, idx)

    km = ~key
    uo = jnp.where(km >= jnp.uint32(0x80000000),
                   km ^ jnp.uint32(0x80000000), ~km)
    ks_ref[...] = lax.bitcast_convert_type(uo, jnp.float32)
    is_ref[...] = idx


def _run_sort(score_t):
    return pl.pallas_call(
        _sort_body,
        out_shape=[
            jax.ShapeDtypeStruct((_N, _B), jnp.float32),
            jax.ShapeDtypeStruct((_N, _B), jnp.int32),
        ],
    )(score_t)


def kernel(input_x, mask, ln_g, ln_b, W1, b1, W2, b2, W3, b3, W4, b4):
    B, H, W_, C = input_x.shape
    N = H * W_
    gnoise = jax.random.gumbel(jax.random.key(42), (B, N, 2),
                               dtype=jnp.float32)
    gnoise_t = jnp.swapaxes(gnoise, 1, 2)
    xr = input_x.reshape(B, N, C)
    mr = mask.reshape(B, N, 1)
    score_row, nm_row = _run_mlp(xr, mr, gnoise_t, ln_g, ln_b,
                                 W1, b1, W2, b2, W3, b3, W4, b4)
    score = score_row.reshape(B, N)
    new_mask = nm_row.reshape(B, H, W_, 1)
    sorted_score_t, sorted_idx_t = _run_sort(score.T)
    sorted_score = sorted_score_t.T
    sorted_idx = sorted_idx_t.T
    keep_score = sorted_score[:, :_KEEP]
    drop_score = sorted_score[:, _KEEP:]
    keep_idx = sorted_idx[:, :_KEEP]
    drop_idx = sorted_idx[:, _KEEP:]
    return (keep_score, drop_score, keep_idx, drop_idx, new_mask,
            score.reshape(B, H, W_))


# speedup vs baseline: 1.0104x; 1.0104x over previous
import jax
import jax.numpy as jnp
from jax import lax
from jax.experimental import pallas as pl

_B, _H, _W, _C = 128, 32, 32, 96
_N = _H * _W
_KEEP = _N // 2
_RECIP96 = 0.010416666977107525

_ERF_P = (7.85386146e-05, -0.000801019371, 0.00518832775, -0.0268538129,
          0.112835854, -0.37612626, 1.12837911)
_ERFC_P = (0.0232682, -0.138703942, 0.368742466, -0.582473278, 0.621000469,
           -0.494451523, 0.340488, -0.274112701, 0.563825965)
_ERFC_R = (-10.477664, 12.9772, -7.49551868, 2.92101908, -1.01526523,
           0.42184633, -0.282076746, 0.564189494)


def _horner(r, coeffs):
    p = r * coeffs[0] + coeffs[1]
    for c in coeffs[2:]:
        p = p * r + c
    return p


def _erfc_xla(x):
    ax = jnp.abs(x)
    x2 = x * x
    small = 1.0 - x * _horner(x2, _ERF_P)
    nx2 = -x2
    z = jnp.exp(nx2)
    zq = z * (1.0 / ax)
    r = 1.0 / x2
    p = jnp.where(ax < 2.0, _horner(r, _ERFC_P), _horner(r, _ERFC_R))
    y = zq * p
    y = jnp.where(nx2 < -88.7228394, 0.0, y)
    y = jnp.where(x < 0.0, 2.0 - y, y)
    return jnp.where(ax < 1.0, small, y)


def _gelu_exact(x):
    return (0.5 * x) * _erfc_xla(-x * 0.7071067811865476)


def _tree96(v):
    gs = []
    for g in range(3):
        a = v[32 * g: 32 * g + 8]
        for r in range(1, 4):
            a = a + v[32 * g + 8 * r: 32 * g + 8 * r + 8]
        u = a[0:4] + a[4:8]
        w = u[0:2] + u[2:4]
        gs.append(w[0:1] + w[1:2])
    return (gs[0] + gs[1]) + gs[2]


def _make_mlp_body():
    def body(x_ref, m_ref, g_ref, lng_ref, lnb_ref, w1_ref, b1_ref, w2_ref,
             b2_ref, w3_ref, b3_ref, w4_ref, b4_ref, score_ref, nm_ref):
        x = x_ref[0] * m_ref[0]
        xt = jnp.transpose(x)
        mu = _tree96(xt) * _RECIP96
        cent = xt - mu
        var = _tree96(cent * cent) * _RECIP96
        y = cent / jnp.sqrt(var + 1e-5) * lng_ref[...] + lnb_ref[...]
        h1 = _gelu_exact(jnp.dot(w1_ref[...], y) + b1_ref[...])
        gin = h1[_C // 2:, :]
        acc = gin[:, 0:128]
        for ch in range(1, 8):
            acc = acc + gin[:, 128 * ch: 128 * (ch + 1)]
        gm = jnp.sum(acc, axis=1, keepdims=True) * (1.0 / 1024.0)
        xcat = jnp.concatenate(
            [h1[: _C // 2, :], jnp.broadcast_to(gm, (_C // 2, _N))], axis=0)
        h2 = _gelu_exact(jnp.dot(w2_ref[...], xcat) + b2_ref[...])
        h3 = _gelu_exact(jnp.dot(w3_ref[...], h2) + b3_ref[...])
        logits = jnp.dot(w4_ref[...], h3) + b4_ref[...]
        mx = jnp.maximum(logits[0:1], logits[1:2])
        sh = logits - mx
        e = jnp.exp(sh)
        lse = jnp.log(e[0:1] + e[1:2])
        pred = sh - lse
        score_ref[0] = pred[0:1]
        vv = pred + g_ref[0]
        mv = jnp.maximum(vv[0:1], vv[1:2])
        e2 = jnp.exp(vv - mv)
        nm_ref[0] = e2[0:1] / (e2[0:1] + e2[1:2])
    return body


def _run_mlp(xr, mr, gnoise_t, ln_g, ln_b, W1, b1, W2, b2, W3, b3, W4, b4):
    row3 = lambda b: (b, 0, 0)
    full = lambda b: (0, 0)
    return pl.pallas_call(
        _make_mlp_body(),
        grid=(_B,),
        in_specs=[
            pl.BlockSpec((1, _N, _C), row3),
            pl.BlockSpec((1, _N, 1), row3),
            pl.BlockSpec((1, 2, _N), row3),
            pl.BlockSpec((_C, 1), full),
            pl.BlockSpec((_C, 1), full),
            pl.BlockSpec((_C, _C), full),
            pl.BlockSpec((_C, 1), full),
            pl.BlockSpec((_C // 2, _C), full),
            pl.BlockSpec((_C // 2, 1), full),
            pl.BlockSpec((_C // 4, _C // 2), full),
            pl.BlockSpec((_C // 4, 1), full),
            pl.BlockSpec((2, _C // 4), full),
            pl.BlockSpec((2, 1), full),
        ],
        out_specs=[
            pl.BlockSpec((1, 1, _N), row3),
            pl.BlockSpec((1, 1, _N), row3),
        ],
        out_shape=[
            jax.ShapeDtypeStruct((_B, 1, _N), jnp.float32),
            jax.ShapeDtypeStruct((_B, 1, _N), jnp.float32),
        ],
    )(xr, mr, gnoise_t,
      ln_g.reshape(_C, 1), ln_b.reshape(_C, 1),
      W1, b1.reshape(_C, 1),
      W2, b2.reshape(_C // 2, 1),
      W3, b3.reshape(_C // 4, 1),
      W4, b4.reshape(2, 1))


def _sort_body(s_ref, ks_ref, is_ref):
    s = jnp.transpose(s_ref[...])
    u = lax.bitcast_convert_type(s, jnp.uint32)
    kmono = jnp.where(u >= jnp.uint32(0x80000000), ~u,
                      u | jnp.uint32(0x80000000))
    key = ~kmono
    idx = lax.broadcasted_iota(jnp.int32, (_N, _B), 0)
    row = lax.broadcasted_iota(jnp.int32, (_N, 1), 0)

    def roll0(x, sh):
        sh = sh % _N
        return jnp.concatenate([x[sh:], x[:sh]], axis=0)

    for stage in range(10):
        for sub in range(stage, -1, -1):
            d = 1 << sub
            dir_up = (row & (2 << stage)) == 0
            if d >= 8:
                g = _N // (2 * d)
                ka = key.reshape(g, 2, d, _B)
                ia = idx.reshape(g, 2, d, _B)
                k0, k1 = ka[:, 0], ka[:, 1]
                i0, i1 = ia[:, 0], ia[:, 1]
                up = dir_up.reshape(g, 2, d, 1)[:, 0]
                cmp = (k0 > k1) | ((k0 == k1) & (i0 > i1))
                swap = jnp.logical_not(cmp ^ up)
                nk0 = jnp.where(swap, k1, k0)
                nk1 = jnp.where(swap, k0, k1)
                ni0 = jnp.where(swap, i1, i0)
                ni1 = jnp.where(swap, i0, i1)
                key = jnp.concatenate(
                    [nk0[:, None], nk1[:, None]], axis=1).reshape(_N, _B)
                idx = jnp.concatenate(
                    [ni0[:, None], ni1[:, None]], axis=1).reshape(_N, _B)
            else:
                self_lo = (row & d) == 0
                kp = jnp.where(self_lo, roll0(key, d), roll0(key, -d))
                ip = jnp.where(self_lo, roll0(idx, d), roll0(idx, -d))
                cmp = (key > kp) | ((key == kp) & (idx > ip))
                g_pair = cmp ^ (~self_lo)
                swap = jnp.logical_not(g_pair ^ dir_up)
                key = jnp.where(swap, kp, key)
                idx = jnp.where(swap, ip, idx)

    km = ~key
    uo = jnp.where(km >= jnp.uint32(0x80000000),
                   km ^ jnp.uint32(0x80000000), ~km)
    ks_ref[...] = jnp.transpose(lax.bitcast_convert_type(uo, jnp.float32))
    is_ref[...] = jnp.transpose(idx)


def _run_sort(score):
    return pl.pallas_call(
        _sort_body,
        out_shape=[
            jax.ShapeDtypeStruct((_B, _N), jnp.float32),
            jax.ShapeDtypeStruct((_B, _N), jnp.int32),
        ],
    )(score)


def kernel(input_x, mask, ln_g, ln_b, W1, b1, W2, b2, W3, b3, W4, b4):
    B, H, W_, C = input_x.shape
    N = H * W_
    gnoise = jax.random.gumbel(jax.random.key(42), (B, N, 2),
                               dtype=jnp.float32)
    gnoise_t = jnp.swapaxes(gnoise, 1, 2)
    xr = input_x.reshape(B, N, C)
    mr = mask.reshape(B, N, 1)
    score_row, nm_row = _run_mlp(xr, mr, gnoise_t, ln_g, ln_b,
                                 W1, b1, W2, b2, W3, b3, W4, b4)
    score = score_row.reshape(B, N)
    new_mask = nm_row.reshape(B, H, W_, 1)
    sorted_score, sorted_idx = _run_sort(score)
    keep_score = sorted_score[:, :_KEEP]
    drop_score = sorted_score[:, _KEEP:]
    keep_idx = sorted_idx[:, :_KEEP]
    drop_idx = sorted_idx[:, _KEEP:]
    return (keep_score, drop_score, keep_idx, drop_idx, new_mask,
            score.reshape(B, H, W_))
